# baseline (speedup 1.0000x reference)
import functools

import jax
import jax.numpy as jnp
from jax import lax
from jax.experimental import pallas as pl
from jax.experimental.pallas import tpu as pltpu
from jax.experimental.pallas import tpu_sc as plsc

N = 10000
E = 320000
D = 128

NC = 2
NS = 16
NW = NC * NS
EW = E // NW
C = 80
NCH = EW // C
N_PAD = 10240
RS = N_PAD // NS
PR = 80
NP = RS // PR
DW = 16

_mesh = plsc.VectorSubcoreMesh(core_axis_name="c", subcore_axis_name="s")


def _zero_vmem(buf, rows, width):
    per_row = width // 16

    def body(t, _):
        i = t // per_row
        j = (t % per_row) * 16
        buf[i, pl.ds(j, 16)] = jnp.zeros((16,), jnp.float32)
        return 0

    lax.fori_loop(0, rows * per_row, body, 0)


@functools.partial(
    pl.kernel,
    out_type=jax.ShapeDtypeStruct((NC, N_PAD, DW), jnp.float32),
    mesh=_mesh,
    scratch_types=[
        pltpu.VMEM((NCH, C), jnp.int32),
        pltpu.VMEM((C, DW), jnp.float32),
        pltpu.VMEM((PR, DW), jnp.float32),
        pltpu.VMEM_SHARED((N_PAD, DW), jnp.float32),
    ],
)
def _sc_deg(row_hbm, out_hbm, row_v, ones_v, buf_v, acc_sh):
    cid = lax.axis_index("c")
    sid = lax.axis_index("s")
    wid = sid * NC + cid

    pltpu.sync_copy(row_hbm.at[wid], row_v)

    def fill_ones(i, _):
        ones_v[i, :] = jnp.ones((DW,), jnp.float32)
        return 0

    lax.fori_loop(0, C, fill_ones, 0)

    _zero_vmem(buf_v, PR, DW)

    def zpiece(t, _):
        off = pl.multiple_of(sid * RS + t * PR, PR)
        pltpu.sync_copy(buf_v, acc_sh.at[pl.ds(off, PR)])
        return 0

    lax.fori_loop(0, NP, zpiece, 0)
    plsc.subcore_barrier()

    def chunk(k, _):
        pltpu.sync_copy(ones_v, acc_sh.at[row_v.at[k]], add=True)
        return 0

    lax.fori_loop(0, NCH, chunk, 0)
    plsc.subcore_barrier()

    def wpiece(t, _):
        off = pl.multiple_of(sid * RS + t * PR, PR)
        pltpu.sync_copy(acc_sh.at[pl.ds(off, PR)], buf_v)
        pltpu.sync_copy(buf_v, out_hbm.at[cid, pl.ds(off, PR)])
        return 0

    lax.fori_loop(0, NP, wpiece, 0)


@functools.partial(
    pl.kernel,
    out_type=jax.ShapeDtypeStruct((NC, N_PAD, D), jnp.float32),
    mesh=_mesh,
    scratch_types=[
        pltpu.VMEM((EW,), jnp.int32),
        pltpu.VMEM((EW,), jnp.int32),
        pltpu.VMEM((C, D), jnp.float32),
        pltpu.VMEM((C, D), jnp.float32),
        pltpu.VMEM_SHARED((N_PAD, D), jnp.float32),
        pltpu.SemaphoreType.DMA,
        pltpu.SemaphoreType.DMA,
    ],
)
def _sc_msg(h_hbm, row_hbm, col_hbm, out_hbm,
            row_v, col_v, rows_a, rows_b, acc_sh, sem_a, sem_b):
    cid = lax.axis_index("c")
    sid = lax.axis_index("s")
    wid = sid * NC + cid

    pltpu.sync_copy(row_hbm.at[wid], row_v)
    pltpu.sync_copy(col_hbm.at[wid], col_v)

    _zero_vmem(rows_a, PR, D)

    def zpiece(t, _):
        off = pl.multiple_of(sid * RS + t * PR, PR)
        pltpu.sync_copy(rows_a.at[pl.ds(0, PR)], acc_sh.at[pl.ds(off, PR)])
        return 0

    lax.fori_loop(0, NP, zpiece, 0)
    plsc.subcore_barrier()

    def ridx(k):
        return row_v.at[pl.ds(pl.multiple_of(k * C, 8), C)]

    def cidx(k):
        return col_v.at[pl.ds(pl.multiple_of(k * C, 8), C)]

    def pair(kk, _):
        k = 2 * kk
        ga = pltpu.async_copy(h_hbm.at[ridx(k)], rows_a, sem_a)
        gb = pltpu.async_copy(h_hbm.at[ridx(k + 1)], rows_b, sem_b)
        ga.wait()
        pltpu.sync_copy(rows_a, acc_sh.at[cidx(k)], add=True)
        gb.wait()
        pltpu.sync_copy(rows_b, acc_sh.at[cidx(k + 1)], add=True)
        return 0

    lax.fori_loop(0, NCH // 2, pair, 0)
    pltpu.async_copy(h_hbm.at[ridx(NCH - 1)], rows_a, sem_a).wait()
    pltpu.sync_copy(rows_a, acc_sh.at[cidx(NCH - 1)], add=True)
    plsc.subcore_barrier()

    def wpiece(t, _):
        off = pl.multiple_of(sid * RS + t * PR, PR)
        pltpu.sync_copy(acc_sh.at[pl.ds(off, PR)], rows_a.at[pl.ds(0, PR)])
        pltpu.sync_copy(rows_a.at[pl.ds(0, PR)], out_hbm.at[cid, pl.ds(off, PR)])
        return 0

    lax.fori_loop(0, NP, wpiece, 0)


def _l0_body(x_ref, w_ref, b_ref, d0_ref, d1_ref, o_ref):
    dis = lax.rsqrt(d0_ref[...] + d1_ref[...])
    h = lax.dot_general(x_ref[...], w_ref[...], (((1,), (1,)), ((), ())),
                        precision=lax.Precision.HIGHEST)
    o_ref[...] = (h + b_ref[...]) * dis


def _l1_body(p0_ref, p1_ref, w_ref, b_ref, d0_ref, d1_ref, o_ref):
    dis = lax.rsqrt(d0_ref[...] + d1_ref[...])
    u = (p0_ref[...] + p1_ref[...]) * dis
    h = lax.dot_general(u, w_ref[...], (((1,), (1,)), ((), ())),
                        precision=lax.Precision.HIGHEST)
    o_ref[...] = (h + b_ref[...]) * dis


def _fin_body(p0_ref, p1_ref, d0_ref, d1_ref, o_ref):
    dis = lax.rsqrt(d0_ref[...] + d1_ref[...])
    o_ref[...] = (p0_ref[...] + p1_ref[...]) * dis


_out_nd = jax.ShapeDtypeStruct((N, D), jnp.float32)
_tc_l0 = pl.pallas_call(_l0_body, out_shape=_out_nd)
_tc_l1 = pl.pallas_call(_l1_body, out_shape=_out_nd)
_tc_fin = pl.pallas_call(_fin_body, out_shape=_out_nd)


def kernel(x, edge_index, W0, b0, W1, b1):
    row2 = edge_index[0].astype(jnp.int32).reshape(NW, EW)
    col2 = edge_index[1].astype(jnp.int32).reshape(NW, EW)
    row3 = edge_index[0].astype(jnp.int32).reshape(NW, NCH, C)
    b0r = b0.reshape(1, D)
    b1r = b1.reshape(1, D)

    degp = _sc_deg(row3)
    d0 = degp[0, :N, 0:1]
    d1 = degp[1, :N, 0:1]

    h0 = _tc_l0(x, W0, b0r, d0, d1)
    p0 = _sc_msg(h0, row2, col2)
    h1 = _tc_l1(p0[0, :N], p0[1, :N], W1, b1r, d0, d1)
    p1 = _sc_msg(h1, row2, col2)
    return _tc_fin(p1[0, :N], p1[1, :N], d0, d1)


# speedup vs baseline: 1.0281x; 1.0281x over previous
import functools

import jax
import jax.numpy as jnp
from jax import lax
from jax.experimental import pallas as pl
from jax.experimental.pallas import tpu as pltpu
from jax.experimental.pallas import tpu_sc as plsc

N = 10000
E = 320000
D = 128

NC = 2
NS = 16
NW = NC * NS
EW = E // NW
C = 80
NCH = EW // C
N_PAD = 10240
RS = N_PAD // NS
PR = 80
NP = RS // PR
DW = 16

_mesh = plsc.VectorSubcoreMesh(core_axis_name="c", subcore_axis_name="s")


def _zero_vmem(buf, rows, width):
    per_row = width // 16

    def body(t, _):
        i = t // per_row
        j = (t % per_row) * 16
        buf[i, pl.ds(j, 16)] = jnp.zeros((16,), jnp.float32)
        return 0

    lax.fori_loop(0, rows * per_row, body, 0)


@functools.partial(
    pl.kernel,
    out_type=jax.ShapeDtypeStruct((NC, N_PAD, DW), jnp.float32),
    mesh=_mesh,
    scratch_types=[
        pltpu.VMEM((NCH, C), jnp.int32),
        pltpu.VMEM((C, DW), jnp.float32),
        pltpu.VMEM((PR, DW), jnp.float32),
        pltpu.VMEM_SHARED((N_PAD, DW), jnp.float32),
    ],
)
def _sc_deg(row_hbm, out_hbm, row_v, ones_v, buf_v, acc_sh):
    cid = lax.axis_index("c")
    sid = lax.axis_index("s")
    wid = sid * NC + cid

    pltpu.sync_copy(row_hbm.at[wid], row_v)

    def fill_ones(i, _):
        ones_v[i, :] = jnp.ones((DW,), jnp.float32)
        return 0

    lax.fori_loop(0, C, fill_ones, 0)

    _zero_vmem(buf_v, PR, DW)

    def zpiece(t, _):
        off = pl.multiple_of(sid * RS + t * PR, PR)
        pltpu.sync_copy(buf_v, acc_sh.at[pl.ds(off, PR)])
        return 0

    lax.fori_loop(0, NP, zpiece, 0)
    plsc.subcore_barrier()

    def chunk(k, _):
        pltpu.sync_copy(ones_v, acc_sh.at[row_v.at[k]], add=True)
        return 0

    lax.fori_loop(0, NCH, chunk, 0)
    plsc.subcore_barrier()

    def wpiece(t, _):
        off = pl.multiple_of(sid * RS + t * PR, PR)
        pltpu.sync_copy(acc_sh.at[pl.ds(off, PR)], buf_v)
        pltpu.sync_copy(buf_v, out_hbm.at[cid, pl.ds(off, PR)])
        return 0

    lax.fori_loop(0, NP, wpiece, 0)


@functools.partial(
    pl.kernel,
    out_type=jax.ShapeDtypeStruct((NC, N_PAD, D), jnp.float32),
    mesh=_mesh,
    scratch_types=[
        pltpu.VMEM((EW,), jnp.int32),
        pltpu.VMEM((EW,), jnp.int32),
        pltpu.VMEM((C, D), jnp.float32),
        pltpu.VMEM((C, D), jnp.float32),
        pltpu.VMEM_SHARED((N_PAD, D), jnp.float32),
        pltpu.SemaphoreType.DMA,
        pltpu.SemaphoreType.DMA,
        pltpu.SemaphoreType.DMA,
        pltpu.SemaphoreType.DMA,
    ],
)
def _sc_msg(h_hbm, row_hbm, col_hbm, out_hbm,
            row_v, col_v, rows_a, rows_b, acc_sh,
            gsem_a, gsem_b, ssem_a, ssem_b):
    cid = lax.axis_index("c")
    sid = lax.axis_index("s")
    wid = sid * NC + cid

    pltpu.sync_copy(row_hbm.at[wid], row_v)
    pltpu.sync_copy(col_hbm.at[wid], col_v)

    _zero_vmem(rows_a, PR, D)

    def zpiece(t, _):
        off = pl.multiple_of(sid * RS + t * PR, PR)
        pltpu.sync_copy(rows_a.at[pl.ds(0, PR)], acc_sh.at[pl.ds(off, PR)])
        return 0

    lax.fori_loop(0, NP, zpiece, 0)
    plsc.subcore_barrier()

    def ridx(k):
        return row_v.at[pl.ds(pl.multiple_of(k * C, 8), C)]

    def cidx(k):
        return col_v.at[pl.ds(pl.multiple_of(k * C, 8), C)]

    def gather(k, buf, sem):
        pltpu.async_copy(h_hbm.at[ridx(lax.rem(k, NCH))], buf, sem)

    def scatter(k, buf, sem):
        pltpu.async_copy(buf, acc_sh.at[cidx(k)], sem, add=True)

    def wait(buf, sem):
        pltpu.make_async_copy(h_hbm.at[ridx(0)], buf, sem).wait()

    gather(0, rows_a, gsem_a)
    gather(1, rows_b, gsem_b)

    def pair(kk, _):
        k = 2 * kk
        wait(rows_a, gsem_a)
        scatter(k, rows_a, ssem_a)
        wait(rows_b, gsem_b)
        scatter(k + 1, rows_b, ssem_b)
        wait(rows_a, ssem_a)
        gather(k + 2, rows_a, gsem_a)
        wait(rows_b, ssem_b)
        gather(k + 3, rows_b, gsem_b)
        return 0

    lax.fori_loop(0, NCH // 2, pair, 0)
    wait(rows_a, gsem_a)
    pltpu.sync_copy(rows_a, acc_sh.at[cidx(NCH - 1)], add=True)
    wait(rows_b, gsem_b)
    plsc.subcore_barrier()

    def wpiece(t, _):
        off = pl.multiple_of(sid * RS + t * PR, PR)
        pltpu.sync_copy(acc_sh.at[pl.ds(off, PR)], rows_a.at[pl.ds(0, PR)])
        pltpu.sync_copy(rows_a.at[pl.ds(0, PR)], out_hbm.at[cid, pl.ds(off, PR)])
        return 0

    lax.fori_loop(0, NP, wpiece, 0)


def _l0_body(x_ref, w_ref, b_ref, d0_ref, d1_ref, o_ref):
    dis = lax.rsqrt(d0_ref[...] + d1_ref[...])
    h = lax.dot_general(x_ref[...], w_ref[...], (((1,), (1,)), ((), ())),
                        precision=lax.Precision.HIGHEST)
    o_ref[...] = (h + b_ref[...]) * dis


def _l1_body(p0_ref, p1_ref, w_ref, b_ref, d0_ref, d1_ref, o_ref):
    dis = lax.rsqrt(d0_ref[...] + d1_ref[...])
    u = (p0_ref[...] + p1_ref[...]) * dis
    h = lax.dot_general(u, w_ref[...], (((1,), (1,)), ((), ())),
                        precision=lax.Precision.HIGHEST)
    o_ref[...] = (h + b_ref[...]) * dis


def _fin_body(p0_ref, p1_ref, d0_ref, d1_ref, o_ref):
    dis = lax.rsqrt(d0_ref[...] + d1_ref[...])
    o_ref[...] = (p0_ref[...] + p1_ref[...]) * dis


_out_nd = jax.ShapeDtypeStruct((N, D), jnp.float32)
_tc_l0 = pl.pallas_call(_l0_body, out_shape=_out_nd)
_tc_l1 = pl.pallas_call(_l1_body, out_shape=_out_nd)
_tc_fin = pl.pallas_call(_fin_body, out_shape=_out_nd)


def kernel(x, edge_index, W0, b0, W1, b1):
    row2 = edge_index[0].astype(jnp.int32).reshape(NW, EW)
    col2 = edge_index[1].astype(jnp.int32).reshape(NW, EW)
    row3 = edge_index[0].astype(jnp.int32).reshape(NW, NCH, C)
    b0r = b0.reshape(1, D)
    b1r = b1.reshape(1, D)

    degp = _sc_deg(row3)
    d0 = degp[0, :N, 0:1]
    d1 = degp[1, :N, 0:1]

    h0 = _tc_l0(x, W0, b0r, d0, d1)
    p0 = _sc_msg(h0, row2, col2)
    h1 = _tc_l1(p0[0, :N], p0[1, :N], W1, b1r, d0, d1)
    p1 = _sc_msg(h1, row2, col2)
    return _tc_fin(p1[0, :N], p1[1, :N], d0, d1)


# speedup vs baseline: 1.1295x; 1.0986x over previous
import functools

import jax
import jax.numpy as jnp
from jax import lax
from jax.experimental import pallas as pl
from jax.experimental.pallas import tpu as pltpu
from jax.experimental.pallas import tpu_sc as plsc

N = 10000
E = 320000
D = 128

NC = 2
NS = 16
NW = NC * NS
EW = E // NW
C = 40
NCH = EW // C
CD = 80
NCHD = EW // CD
N_PAD = 10240
RS = N_PAD // NS
NP2 = RS // C // 2
DW = 16

_mesh = plsc.VectorSubcoreMesh(core_axis_name="c", subcore_axis_name="s")


def _zero_vmem(buf, rows, width):
    per_row = width // 16

    def body(t, _):
        i = t // per_row
        j = (t % per_row) * 16
        buf[i, pl.ds(j, 16)] = jnp.zeros((16,), jnp.float32)
        return 0

    lax.fori_loop(0, rows * per_row, body, 0)


@functools.partial(
    pl.kernel,
    out_type=jax.ShapeDtypeStruct((NC, N_PAD, DW), jnp.float32),
    mesh=_mesh,
    scratch_types=[
        pltpu.VMEM((NCHD, CD), jnp.int32),
        pltpu.VMEM((CD, DW), jnp.float32),
        pltpu.VMEM((CD, DW), jnp.float32),
        pltpu.VMEM_SHARED((N_PAD, DW), jnp.float32),
        pltpu.SemaphoreType.DMA,
        pltpu.SemaphoreType.DMA,
    ],
)
def _sc_deg(row_hbm, out_hbm, row_v, ones_v, buf_v, acc_sh, sem_a, sem_b):
    cid = lax.axis_index("c")
    sid = lax.axis_index("s")
    wid = sid * NC + cid

    pltpu.sync_copy(row_hbm.at[wid], row_v)

    def fill_ones(i, _):
        ones_v[i, :] = jnp.ones((DW,), jnp.float32)
        return 0

    lax.fori_loop(0, CD, fill_ones, 0)

    _zero_vmem(buf_v, CD, DW)

    def zpiece(t, _):
        off = pl.multiple_of(sid * RS + t * CD, 8)
        pltpu.sync_copy(buf_v, acc_sh.at[pl.ds(off, CD)])
        return 0

    lax.fori_loop(0, RS // CD, zpiece, 0)
    plsc.subcore_barrier()

    def batch(kk, _):
        k = 2 * kk
        da = pltpu.async_copy(ones_v, acc_sh.at[row_v.at[k]], sem_a,
                              add=True)
        db = pltpu.async_copy(ones_v, acc_sh.at[row_v.at[k + 1]], sem_b,
                              add=True)
        da.wait()
        db.wait()
        return 0

    lax.fori_loop(0, NCHD // 2, batch, 0)
    pltpu.sync_copy(ones_v, acc_sh.at[row_v.at[NCHD - 1]], add=True)
    plsc.subcore_barrier()

    def wpiece(t, _):
        off = pl.multiple_of(sid * RS + t * CD, 8)
        pltpu.sync_copy(acc_sh.at[pl.ds(off, CD)], buf_v)
        pltpu.sync_copy(buf_v, out_hbm.at[cid, pl.ds(off, CD)])
        return 0

    lax.fori_loop(0, RS // CD, wpiece, 0)


@functools.partial(
    pl.kernel,
    out_type=jax.ShapeDtypeStruct((NC, N_PAD, D), jnp.float32),
    mesh=_mesh,
    scratch_types=[
        pltpu.VMEM((EW,), jnp.int32),
        pltpu.VMEM((EW,), jnp.int32),
        pltpu.VMEM((C, D), jnp.float32),
        pltpu.VMEM((C, D), jnp.float32),
        pltpu.VMEM((C, D), jnp.float32),
        pltpu.VMEM((C, D), jnp.float32),
        pltpu.VMEM_SHARED((N_PAD, D), jnp.float32),
        pltpu.SemaphoreType.DMA,
        pltpu.SemaphoreType.DMA,
        pltpu.SemaphoreType.DMA,
        pltpu.SemaphoreType.DMA,
        pltpu.SemaphoreType.DMA,
        pltpu.SemaphoreType.DMA,
        pltpu.SemaphoreType.DMA,
        pltpu.SemaphoreType.DMA,
    ],
)
def _sc_msg(h_hbm, row_hbm, col_hbm, out_hbm,
            row_v, col_v, r0, r1, r2, r3, acc_sh,
            g0, g1, g2, g3, s0, s1, s2, s3):
    rows = (r0, r1, r2, r3)
    gsem = (g0, g1, g2, g3)
    ssem = (s0, s1, s2, s3)
    cid = lax.axis_index("c")
    sid = lax.axis_index("s")
    wid = sid * NC + cid

    pltpu.sync_copy(row_hbm.at[wid], row_v)
    pltpu.sync_copy(col_hbm.at[wid], col_v)

    def ridx(k):
        return row_v.at[pl.ds(pl.multiple_of(k * C, 8), C)]

    def cidx(k):
        return col_v.at[pl.ds(pl.multiple_of(k * C, 8), C)]

    _zero_vmem(r0, C, D)

    def zpiece(t, _):
        off = pl.multiple_of(sid * RS + 2 * t * C, 8)
        da = pltpu.async_copy(r0, acc_sh.at[pl.ds(off, C)], g0)
        db = pltpu.async_copy(r0, acc_sh.at[pl.ds(off + C, C)], g1)
        da.wait()
        db.wait()
        return 0

    lax.fori_loop(0, NP2, zpiece, 0)
    plsc.subcore_barrier()

    def stage_a(j, slot):
        pltpu.make_async_copy(h_hbm.at[ridx(0)], rows[slot], gsem[slot]).wait()
        pltpu.async_copy(rows[slot], acc_sh.at[cidx(j)], ssem[slot], add=True)

    def stage_b(j, slot):
        pltpu.make_async_copy(h_hbm.at[ridx(0)], rows[slot], ssem[slot]).wait()
        pltpu.async_copy(h_hbm.at[ridx(j + 4)], rows[slot], gsem[slot])

    for i in range(4):
        pltpu.async_copy(h_hbm.at[ridx(i)], rows[i], gsem[i])
    stage_a(0, 0)
    stage_a(1, 1)
    stage_a(2, 2)
    stage_b(0, 0)
    stage_a(3, 3)
    stage_b(1, 1)

    def ring(m, _):
        k = 4 * m
        stage_a(k, 0)
        stage_b(k - 2, 2)
        stage_a(k + 1, 1)
        stage_b(k - 1, 3)
        stage_a(k + 2, 2)
        stage_b(k, 0)
        stage_a(k + 3, 3)
        stage_b(k + 1, 1)
        return 0

    lax.fori_loop(1, NCH // 4, ring, 0)

    def drain_s(slot):
        pltpu.make_async_copy(h_hbm.at[ridx(0)], rows[slot], ssem[slot]).wait()

    stage_a(NCH - 2, 0)
    drain_s(2)
    stage_a(NCH - 1, 1)
    drain_s(3)
    drain_s(0)
    drain_s(1)
    plsc.subcore_barrier()

    def wpiece(t, _):
        off = pl.multiple_of(sid * RS + 2 * t * C, 8)
        da = pltpu.async_copy(acc_sh.at[pl.ds(off, C)], r0, g0)
        db = pltpu.async_copy(acc_sh.at[pl.ds(off + C, C)], r1, g1)
        da.wait()
        ea = pltpu.async_copy(r0, out_hbm.at[cid, pl.ds(off, C)], s0)
        db.wait()
        eb = pltpu.async_copy(r1, out_hbm.at[cid, pl.ds(off + C, C)], s1)
        ea.wait()
        eb.wait()
        return 0

    lax.fori_loop(0, NP2, wpiece, 0)


def _l0_body(x_ref, w_ref, b_ref, d0_ref, d1_ref, o_ref):
    dis = lax.rsqrt(d0_ref[...] + d1_ref[...])
    h = lax.dot_general(x_ref[...], w_ref[...], (((1,), (1,)), ((), ())),
                        precision=lax.Precision.HIGHEST)
    o_ref[...] = (h + b_ref[...]) * dis


def _l1_body(p0_ref, p1_ref, w_ref, b_ref, d0_ref, d1_ref, o_ref):
    dis = lax.rsqrt(d0_ref[...] + d1_ref[...])
    u = (p0_ref[...] + p1_ref[...]) * dis
    h = lax.dot_general(u, w_ref[...], (((1,), (1,)), ((), ())),
                        precision=lax.Precision.HIGHEST)
    o_ref[...] = (h + b_ref[...]) * dis


def _fin_body(p0_ref, p1_ref, d0_ref, d1_ref, o_ref):
    dis = lax.rsqrt(d0_ref[...] + d1_ref[...])
    o_ref[...] = (p0_ref[...] + p1_ref[...]) * dis


_out_nd = jax.ShapeDtypeStruct((N, D), jnp.float32)
_tc_l0 = pl.pallas_call(_l0_body, out_shape=_out_nd)
_tc_l1 = pl.pallas_call(_l1_body, out_shape=_out_nd)
_tc_fin = pl.pallas_call(_fin_body, out_shape=_out_nd)


def kernel(x, edge_index, W0, b0, W1, b1):
    row2 = edge_index[0].astype(jnp.int32).reshape(NW, EW)
    col2 = edge_index[1].astype(jnp.int32).reshape(NW, EW)
    row3 = edge_index[0].astype(jnp.int32).reshape(NW, NCHD, CD)
    b0r = b0.reshape(1, D)
    b1r = b1.reshape(1, D)

    degp = _sc_deg(row3)
    d0 = degp[0, :N, 0:1]
    d1 = degp[1, :N, 0:1]

    h0 = _tc_l0(x, W0, b0r, d0, d1)
    p0 = _sc_msg(h0, row2, col2)
    h1 = _tc_l1(p0[0, :N], p0[1, :N], W1, b1r, d0, d1)
    p1 = _sc_msg(h1, row2, col2)
    return _tc_fin(p1[0, :N], p1[1, :N], d0, d1)


# speedup vs baseline: 1.3969x; 1.2367x over previous
import functools

import jax
import jax.numpy as jnp
from jax import lax
from jax.experimental import pallas as pl
from jax.experimental.pallas import tpu as pltpu
from jax.experimental.pallas import tpu_sc as plsc

N = 10000
E = 320000
D = 128

NC = 2
NS = 16
NW = NC * NS
EW = E // NW
C = 80
NCH = EW // C
CD = 80
NCHD = EW // CD
N_PAD = 10240
RS = N_PAD // NS
SLAB = 624
DW = 16

_mesh = plsc.VectorSubcoreMesh(core_axis_name="c", subcore_axis_name="s")


def _zero_vmem(buf, rows, width):
    per_row = width // 16

    def body(t, _):
        i = t // per_row
        j = (t % per_row) * 16
        buf[i, pl.ds(j, 16)] = jnp.zeros((16,), jnp.float32)
        return 0

    lax.fori_loop(0, rows * per_row, body, 0)


@functools.partial(
    pl.kernel,
    out_type=jax.ShapeDtypeStruct((NC, N_PAD, DW), jnp.float32),
    mesh=_mesh,
    scratch_types=[
        pltpu.VMEM((NCHD, CD), jnp.int32),
        pltpu.VMEM((CD, DW), jnp.float32),
        pltpu.VMEM((CD, DW), jnp.float32),
        pltpu.VMEM_SHARED((N_PAD, DW), jnp.float32),
        pltpu.SemaphoreType.DMA,
        pltpu.SemaphoreType.DMA,
    ],
)
def _sc_deg(row_hbm, out_hbm, row_v, ones_v, buf_v, acc_sh, sem_a, sem_b):
    cid = lax.axis_index("c")
    sid = lax.axis_index("s")
    wid = sid * NC + cid

    pltpu.sync_copy(row_hbm.at[wid], row_v)

    def fill_ones(i, _):
        ones_v[i, :] = jnp.ones((DW,), jnp.float32)
        return 0

    lax.fori_loop(0, CD, fill_ones, 0)

    _zero_vmem(buf_v, CD, DW)

    def zpiece(t, _):
        off = pl.multiple_of(sid * RS + t * CD, 8)
        pltpu.sync_copy(buf_v, acc_sh.at[pl.ds(off, CD)])
        return 0

    lax.fori_loop(0, RS // CD, zpiece, 0)
    plsc.subcore_barrier()

    def batch(kk, _):
        k = 2 * kk
        da = pltpu.async_copy(ones_v, acc_sh.at[row_v.at[k]], sem_a,
                              add=True)
        db = pltpu.async_copy(ones_v, acc_sh.at[row_v.at[k + 1]], sem_b,
                              add=True)
        da.wait()
        db.wait()
        return 0

    lax.fori_loop(0, NCHD // 2, batch, 0)
    pltpu.sync_copy(ones_v, acc_sh.at[row_v.at[NCHD - 1]], add=True)
    plsc.subcore_barrier()

    def wpiece(t, _):
        off = pl.multiple_of(sid * RS + t * CD, 8)
        pltpu.sync_copy(acc_sh.at[pl.ds(off, CD)], buf_v)
        pltpu.sync_copy(buf_v, out_hbm.at[cid, pl.ds(off, CD)])
        return 0

    lax.fori_loop(0, RS // CD, wpiece, 0)


@functools.partial(
    pl.kernel,
    out_type=jax.ShapeDtypeStruct((NC, N, D), jnp.float32),
    mesh=_mesh,
    scratch_types=[
        pltpu.VMEM((EW,), jnp.int32),
        pltpu.VMEM((EW,), jnp.int32),
        pltpu.VMEM((C, D), jnp.float32),
        pltpu.VMEM((C, D), jnp.float32),
        pltpu.VMEM((C, D), jnp.float32),
        pltpu.VMEM_SHARED((N, D), jnp.float32),
        pltpu.SemaphoreType.DMA,
        pltpu.SemaphoreType.DMA,
        pltpu.SemaphoreType.DMA,
        pltpu.SemaphoreType.DMA,
        pltpu.SemaphoreType.DMA,
        pltpu.SemaphoreType.DMA,
    ],
)
def _sc_msg(h_hbm, row_hbm, col_hbm, out_hbm,
            row_v, col_v, r0, r1, r2, acc_sh,
            g0, g1, g2, s0, s1, s2):
    rows = (r0, r1, r2)
    gsem = (g0, g1, g2)
    ssem = (s0, s1, s2)
    cid = lax.axis_index("c")
    sid = lax.axis_index("s")
    wid = sid * NC + cid

    pltpu.sync_copy(row_hbm.at[wid], row_v)
    pltpu.sync_copy(col_hbm.at[wid], col_v)

    def ridx(k):
        return row_v.at[pl.ds(pl.multiple_of(k * C, 8), C)]

    def cidx(k):
        return col_v.at[pl.ds(pl.multiple_of(k * C, 8), C)]

    _zero_vmem(r0, C, D)

    def zpiece(t, _):
        off = pl.multiple_of(sid * SLAB + t * 80, 8)
        pltpu.sync_copy(r0, acc_sh.at[pl.ds(off, 80)])
        return 0

    lax.fori_loop(0, 7, zpiece, 0)
    tail = pl.multiple_of(sid * SLAB + 560, 8)

    @pl.when(sid == NS - 1)
    def _():
        pltpu.sync_copy(r0, acc_sh.at[pl.ds(tail, 80)])

    @pl.when(sid < NS - 1)
    def _():
        pltpu.sync_copy(r0.at[pl.ds(0, 64)], acc_sh.at[pl.ds(tail, 64)])

    plsc.subcore_barrier()

    def stage_a(j, slot):
        pltpu.make_async_copy(h_hbm.at[ridx(0)], rows[slot], gsem[slot]).wait()
        pltpu.async_copy(rows[slot], acc_sh.at[cidx(j)], ssem[slot], add=True)

    def stage_b(j, slot):
        pltpu.make_async_copy(h_hbm.at[ridx(0)], rows[slot], ssem[slot]).wait()
        pltpu.async_copy(h_hbm.at[ridx(j + 3)], rows[slot], gsem[slot])

    for i in range(3):
        pltpu.async_copy(h_hbm.at[ridx(i)], rows[i], gsem[i])
    stage_a(0, 0)
    stage_a(1, 1)
    stage_b(0, 0)
    stage_a(2, 2)
    stage_b(1, 1)

    def ring(m, _):
        k = 3 * m
        stage_a(k, 0)
        stage_b(k - 1, 2)
        stage_a(k + 1, 1)
        stage_b(k, 0)
        stage_a(k + 2, 2)
        stage_b(k + 1, 1)
        return 0

    lax.fori_loop(1, 41, ring, 0)

    def drain_s(slot):
        pltpu.make_async_copy(h_hbm.at[ridx(0)], rows[slot], ssem[slot]).wait()

    stage_a(NCH - 2, 0)
    drain_s(2)
    stage_a(NCH - 1, 1)
    drain_s(0)
    drain_s(1)
    plsc.subcore_barrier()

    def wpiece(t, _):
        off = pl.multiple_of(sid * SLAB + t * 80, 8)
        pltpu.sync_copy(acc_sh.at[pl.ds(off, 80)], r0)
        pltpu.sync_copy(r0, out_hbm.at[cid, pl.ds(off, 80)])
        return 0

    lax.fori_loop(0, 7, wpiece, 0)

    @pl.when(sid == NS - 1)
    def _():
        pltpu.sync_copy(acc_sh.at[pl.ds(tail, 80)], r1)
        pltpu.sync_copy(r1, out_hbm.at[cid, pl.ds(tail, 80)])

    @pl.when(sid < NS - 1)
    def _():
        pltpu.sync_copy(acc_sh.at[pl.ds(tail, 64)], r1.at[pl.ds(0, 64)])
        pltpu.sync_copy(r1.at[pl.ds(0, 64)], out_hbm.at[cid, pl.ds(tail, 64)])


def _l0_body(x_ref, w_ref, b_ref, d0_ref, d1_ref, o_ref):
    dis = lax.rsqrt(d0_ref[...] + d1_ref[...])
    h = lax.dot_general(x_ref[...], w_ref[...], (((1,), (1,)), ((), ())),
                        precision=lax.Precision.HIGHEST)
    o_ref[...] = (h + b_ref[...]) * dis


def _l1_body(p0_ref, p1_ref, w_ref, b_ref, d0_ref, d1_ref, o_ref):
    dis = lax.rsqrt(d0_ref[...] + d1_ref[...])
    u = (p0_ref[...] + p1_ref[...]) * dis
    h = lax.dot_general(u, w_ref[...], (((1,), (1,)), ((), ())),
                        precision=lax.Precision.HIGHEST)
    o_ref[...] = (h + b_ref[...]) * dis


def _fin_body(p0_ref, p1_ref, d0_ref, d1_ref, o_ref):
    dis = lax.rsqrt(d0_ref[...] + d1_ref[...])
    o_ref[...] = (p0_ref[...] + p1_ref[...]) * dis


_out_nd = jax.ShapeDtypeStruct((N, D), jnp.float32)
_tc_l0 = pl.pallas_call(_l0_body, out_shape=_out_nd)
_tc_l1 = pl.pallas_call(_l1_body, out_shape=_out_nd)
_tc_fin = pl.pallas_call(_fin_body, out_shape=_out_nd)


def kernel(x, edge_index, W0, b0, W1, b1):
    row2 = edge_index[0].astype(jnp.int32).reshape(NW, EW)
    col2 = edge_index[1].astype(jnp.int32).reshape(NW, EW)
    row3 = edge_index[0].astype(jnp.int32).reshape(NW, NCHD, CD)
    b0r = b0.reshape(1, D)
    b1r = b1.reshape(1, D)

    degp = _sc_deg(row3)
    d0 = degp[0, :N, 0:1]
    d1 = degp[1, :N, 0:1]

    h0 = _tc_l0(x, W0, b0r, d0, d1)
    p0 = _sc_msg(h0, row2, col2)
    h1 = _tc_l1(p0[0, :N], p0[1, :N], W1, b1r, d0, d1)
    p1 = _sc_msg(h1, row2, col2)
    return _tc_fin(p1[0, :N], p1[1, :N], d0, d1)


# speedup vs baseline: 1.4131x; 1.0116x over previous
import functools

import jax
import jax.numpy as jnp
from jax import lax
from jax.experimental import pallas as pl
from jax.experimental.pallas import tpu as pltpu
from jax.experimental.pallas import tpu_sc as plsc

N = 10000
E = 320000
D = 128

NC = 2
NS = 16
NW = NC * NS
EW = E // NW
C = 80
NCH = EW // C
CD = 125
NCHD = EW // CD
N_PAD = 10240
RS = N_PAD // NS
SLAB = 624
DW = 16

_mesh = plsc.VectorSubcoreMesh(core_axis_name="c", subcore_axis_name="s")


def _zero_vmem(buf, rows, width):
    per_row = width // 16

    def body(t, _):
        i = t // per_row
        j = (t % per_row) * 16
        buf[i, pl.ds(j, 16)] = jnp.zeros((16,), jnp.float32)
        return 0

    lax.fori_loop(0, rows * per_row, body, 0)


@functools.partial(
    pl.kernel,
    out_type=jax.ShapeDtypeStruct((NC, N_PAD, DW), jnp.float32),
    mesh=_mesh,
    scratch_types=[
        pltpu.VMEM((NCHD, CD), jnp.int32),
        pltpu.VMEM((CD, DW), jnp.float32),
        pltpu.VMEM((CD, DW), jnp.float32),
        pltpu.VMEM_SHARED((N_PAD, DW), jnp.float32),
        pltpu.SemaphoreType.DMA,
        pltpu.SemaphoreType.DMA,
        pltpu.SemaphoreType.DMA,
    ],
)
def _sc_deg(row_hbm, out_hbm, row_v, ones_v, buf_v, acc_sh,
            sem_a, sem_b, sem_c):
    cid = lax.axis_index("c")
    sid = lax.axis_index("s")
    wid = sid * NC + cid

    pltpu.sync_copy(row_hbm.at[wid], row_v)

    def fill_ones(i, _):
        ones_v[i, :] = jnp.ones((DW,), jnp.float32)
        return 0

    lax.fori_loop(0, CD, fill_ones, 0)

    _zero_vmem(buf_v, CD, DW)

    def zpiece(t, _):
        off = pl.multiple_of(sid * RS + t * 80, 8)
        pltpu.sync_copy(buf_v.at[pl.ds(0, 80)], acc_sh.at[pl.ds(off, 80)])
        return 0

    lax.fori_loop(0, RS // 80, zpiece, 0)
    plsc.subcore_barrier()

    def batch(kk, _):
        k = 3 * kk
        da = pltpu.async_copy(ones_v, acc_sh.at[row_v.at[k]], sem_a,
                              add=True)
        db = pltpu.async_copy(ones_v, acc_sh.at[row_v.at[k + 1]], sem_b,
                              add=True)
        dc = pltpu.async_copy(ones_v, acc_sh.at[row_v.at[k + 2]], sem_c,
                              add=True)
        da.wait()
        db.wait()
        dc.wait()
        return 0

    lax.fori_loop(0, NCHD // 3, batch, 0)
    da = pltpu.async_copy(ones_v, acc_sh.at[row_v.at[NCHD - 2]], sem_a,
                          add=True)
    db = pltpu.async_copy(ones_v, acc_sh.at[row_v.at[NCHD - 1]], sem_b,
                          add=True)
    da.wait()
    db.wait()
    plsc.subcore_barrier()

    def wpiece(t, _):
        off = pl.multiple_of(sid * RS + t * 80, 8)
        pltpu.sync_copy(acc_sh.at[pl.ds(off, 80)], buf_v.at[pl.ds(0, 80)])
        pltpu.sync_copy(buf_v.at[pl.ds(0, 80)], out_hbm.at[cid, pl.ds(off, 80)])
        return 0

    lax.fori_loop(0, RS // 80, wpiece, 0)


@functools.partial(
    pl.kernel,
    out_type=jax.ShapeDtypeStruct((NC, N, D), jnp.float32),
    mesh=_mesh,
    scratch_types=[
        pltpu.VMEM((EW,), jnp.int32),
        pltpu.VMEM((EW,), jnp.int32),
        pltpu.VMEM((C, D), jnp.float32),
        pltpu.VMEM((C, D), jnp.float32),
        pltpu.VMEM((C, D), jnp.float32),
        pltpu.VMEM_SHARED((N, D), jnp.float32),
        pltpu.SemaphoreType.DMA,
        pltpu.SemaphoreType.DMA,
        pltpu.SemaphoreType.DMA,
        pltpu.SemaphoreType.DMA,
        pltpu.SemaphoreType.DMA,
        pltpu.SemaphoreType.DMA,
    ],
)
def _sc_msg(h_hbm, row_hbm, col_hbm, out_hbm,
            row_v, col_v, r0, r1, r2, acc_sh,
            g0, g1, g2, s0, s1, s2):
    rows = (r0, r1, r2)
    gsem = (g0, g1, g2)
    ssem = (s0, s1, s2)
    cid = lax.axis_index("c")
    sid = lax.axis_index("s")
    wid = sid * NC + cid

    pltpu.sync_copy(row_hbm.at[wid], row_v)
    pltpu.sync_copy(col_hbm.at[wid], col_v)

    def ridx(k):
        return row_v.at[pl.ds(pl.multiple_of(k * C, 8), C)]

    def cidx(k):
        return col_v.at[pl.ds(pl.multiple_of(k * C, 8), C)]

    _zero_vmem(r0, C, D)

    def zpiece(t, _):
        off = pl.multiple_of(sid * SLAB + t * 80, 8)
        pltpu.sync_copy(r0, acc_sh.at[pl.ds(off, 80)])
        return 0

    lax.fori_loop(0, 7, zpiece, 0)
    tail = pl.multiple_of(sid * SLAB + 560, 8)

    @pl.when(sid == NS - 1)
    def _():
        pltpu.sync_copy(r0, acc_sh.at[pl.ds(tail, 80)])

    @pl.when(sid < NS - 1)
    def _():
        pltpu.sync_copy(r0.at[pl.ds(0, 64)], acc_sh.at[pl.ds(tail, 64)])

    plsc.subcore_barrier()

    def stage_a(j, slot):
        pltpu.make_async_copy(h_hbm.at[ridx(0)], rows[slot], gsem[slot]).wait()
        pltpu.async_copy(rows[slot], acc_sh.at[cidx(j)], ssem[slot], add=True)

    def stage_b(j, slot):
        pltpu.make_async_copy(h_hbm.at[ridx(0)], rows[slot], ssem[slot]).wait()
        pltpu.async_copy(h_hbm.at[ridx(j + 3)], rows[slot], gsem[slot])

    for i in range(3):
        pltpu.async_copy(h_hbm.at[ridx(i)], rows[i], gsem[i])
    stage_a(0, 0)
    stage_a(1, 1)
    stage_b(0, 0)
    stage_a(2, 2)
    stage_b(1, 1)

    def ring(m, _):
        k = 3 * m
        stage_a(k, 0)
        stage_b(k - 1, 2)
        stage_a(k + 1, 1)
        stage_b(k, 0)
        stage_a(k + 2, 2)
        stage_b(k + 1, 1)
        return 0

    lax.fori_loop(1, 41, ring, 0)

    def drain_s(slot):
        pltpu.make_async_copy(h_hbm.at[ridx(0)], rows[slot], ssem[slot]).wait()

    stage_a(NCH - 2, 0)
    drain_s(2)
    stage_a(NCH - 1, 1)
    drain_s(0)
    drain_s(1)
    plsc.subcore_barrier()

    def wpiece(t, _):
        off = pl.multiple_of(sid * SLAB + t * 80, 8)
        pltpu.sync_copy(acc_sh.at[pl.ds(off, 80)], r0)
        pltpu.sync_copy(r0, out_hbm.at[cid, pl.ds(off, 80)])
        return 0

    lax.fori_loop(0, 7, wpiece, 0)

    @pl.when(sid == NS - 1)
    def _():
        pltpu.sync_copy(acc_sh.at[pl.ds(tail, 80)], r1)
        pltpu.sync_copy(r1, out_hbm.at[cid, pl.ds(tail, 80)])

    @pl.when(sid < NS - 1)
    def _():
        pltpu.sync_copy(acc_sh.at[pl.ds(tail, 64)], r1.at[pl.ds(0, 64)])
        pltpu.sync_copy(r1.at[pl.ds(0, 64)], out_hbm.at[cid, pl.ds(tail, 64)])


def _mm_body(x_ref, w_ref, b_ref, o_ref):
    h = lax.dot_general(x_ref[...], w_ref[...], (((1,), (1,)), ((), ())),
                        precision=lax.Precision.HIGHEST)
    o_ref[...] = h + b_ref[...]


def _scale_body(h_ref, d0_ref, d1_ref, o_ref):
    dis = lax.rsqrt(d0_ref[...] + d1_ref[...])
    o_ref[...] = h_ref[...] * dis


def _l1_body(p0_ref, p1_ref, w_ref, b_ref, d0_ref, d1_ref, o_ref):
    dis = lax.rsqrt(d0_ref[...] + d1_ref[...])
    u = (p0_ref[...] + p1_ref[...]) * dis
    h = lax.dot_general(u, w_ref[...], (((1,), (1,)), ((), ())),
                        precision=lax.Precision.HIGHEST)
    o_ref[...] = (h + b_ref[...]) * dis


def _fin_body(p0_ref, p1_ref, d0_ref, d1_ref, o_ref):
    dis = lax.rsqrt(d0_ref[...] + d1_ref[...])
    o_ref[...] = (p0_ref[...] + p1_ref[...]) * dis


_out_nd = jax.ShapeDtypeStruct((N, D), jnp.float32)
_tc_mm = pl.pallas_call(_mm_body, out_shape=_out_nd)
_tc_scale = pl.pallas_call(_scale_body, out_shape=_out_nd)
_tc_l1 = pl.pallas_call(_l1_body, out_shape=_out_nd)
_tc_fin = pl.pallas_call(_fin_body, out_shape=_out_nd)


def kernel(x, edge_index, W0, b0, W1, b1):
    row2 = edge_index[0].astype(jnp.int32).reshape(NW, EW)
    col2 = edge_index[1].astype(jnp.int32).reshape(NW, EW)
    row3 = edge_index[0].astype(jnp.int32).reshape(NW, NCHD, CD)
    b0r = b0.reshape(1, D)
    b1r = b1.reshape(1, D)

    degp = _sc_deg(row3)
    d0 = degp[0, :N, 0:1]
    d1 = degp[1, :N, 0:1]

    mm0 = _tc_mm(x, W0, b0r)
    h0 = _tc_scale(mm0, d0, d1)
    p0 = _sc_msg(h0, row2, col2)
    h1 = _tc_l1(p0[0, :N], p0[1, :N], W1, b1r, d0, d1)
    p1 = _sc_msg(h1, row2, col2)
    return _tc_fin(p1[0, :N], p1[1, :N], d0, d1)


# speedup vs baseline: 1.5136x; 1.0711x over previous
import functools

import jax
import jax.numpy as jnp
from jax import lax
from jax.experimental import pallas as pl
from jax.experimental.pallas import tpu as pltpu
from jax.experimental.pallas import tpu_sc as plsc

N = 10000
E = 320000
D = 128

NC = 2
NS = 16
NW = NC * NS
EW = E // NW
C = 80
NCH = EW // C
CD = 125
NCHD = EW // CD
N_PAD = 10240
RS = N_PAD // NS
SLAB = 624
DW = 16

_mesh = plsc.VectorSubcoreMesh(core_axis_name="c", subcore_axis_name="s")


def _zero_vmem(buf, rows, width):
    per_row = width // 16

    def body(t, _):
        i = t // per_row
        j = (t % per_row) * 16
        buf[i, pl.ds(j, 16)] = jnp.zeros((16,), jnp.float32)
        return 0

    lax.fori_loop(0, rows * per_row, body, 0)


@functools.partial(
    pl.kernel,
    out_type=jax.ShapeDtypeStruct((NC, N_PAD, DW), jnp.float32),
    mesh=_mesh,
    scratch_types=[
        pltpu.VMEM((NCHD, CD), jnp.int32),
        pltpu.VMEM((CD, DW), jnp.float32),
        pltpu.VMEM((CD, DW), jnp.float32),
        pltpu.VMEM_SHARED((N_PAD, DW), jnp.float32),
        pltpu.SemaphoreType.DMA,
        pltpu.SemaphoreType.DMA,
        pltpu.SemaphoreType.DMA,
    ],
)
def _sc_deg(row_hbm, out_hbm, row_v, ones_v, buf_v, acc_sh,
            sem_a, sem_b, sem_c):
    cid = lax.axis_index("c")
    sid = lax.axis_index("s")
    wid = sid * NC + cid

    pltpu.sync_copy(row_hbm.at[wid], row_v)

    def fill_ones(i, _):
        ones_v[i, :] = jnp.ones((DW,), jnp.float32)
        return 0

    lax.fori_loop(0, CD, fill_ones, 0)

    _zero_vmem(buf_v, CD, DW)

    def zpiece(t, _):
        off = pl.multiple_of(sid * RS + t * 80, 8)
        pltpu.sync_copy(buf_v.at[pl.ds(0, 80)], acc_sh.at[pl.ds(off, 80)])
        return 0

    lax.fori_loop(0, RS // 80, zpiece, 0)
    plsc.subcore_barrier()

    def batch(kk, _):
        k = 3 * kk
        da = pltpu.async_copy(ones_v, acc_sh.at[row_v.at[k]], sem_a,
                              add=True)
        db = pltpu.async_copy(ones_v, acc_sh.at[row_v.at[k + 1]], sem_b,
                              add=True)
        dc = pltpu.async_copy(ones_v, acc_sh.at[row_v.at[k + 2]], sem_c,
                              add=True)
        da.wait()
        db.wait()
        dc.wait()
        return 0

    lax.fori_loop(0, NCHD // 3, batch, 0)
    da = pltpu.async_copy(ones_v, acc_sh.at[row_v.at[NCHD - 2]], sem_a,
                          add=True)
    db = pltpu.async_copy(ones_v, acc_sh.at[row_v.at[NCHD - 1]], sem_b,
                          add=True)
    da.wait()
    db.wait()
    plsc.subcore_barrier()

    def wpiece(t, _):
        off = pl.multiple_of(sid * RS + t * 80, 8)
        pltpu.sync_copy(acc_sh.at[pl.ds(off, 80)], buf_v.at[pl.ds(0, 80)])
        pltpu.sync_copy(buf_v.at[pl.ds(0, 80)], out_hbm.at[cid, pl.ds(off, 80)])
        return 0

    lax.fori_loop(0, RS // 80, wpiece, 0)


@functools.partial(
    pl.kernel,
    out_type=jax.ShapeDtypeStruct((NC, N, D), jnp.float32),
    mesh=_mesh,
    scratch_types=[
        pltpu.VMEM((EW,), jnp.int32),
        pltpu.VMEM((EW,), jnp.int32),
        pltpu.VMEM((C, D), jnp.float32),
        pltpu.VMEM((C, D), jnp.float32),
        pltpu.VMEM((C, D), jnp.float32),
        pltpu.VMEM_SHARED((N, D), jnp.float32),
        pltpu.SemaphoreType.DMA,
        pltpu.SemaphoreType.DMA,
        pltpu.SemaphoreType.DMA,
        pltpu.SemaphoreType.DMA,
        pltpu.SemaphoreType.DMA,
        pltpu.SemaphoreType.DMA,
    ],
)
def _sc_msg(h_hbm, row_hbm, col_hbm, out_hbm,
            row_v, col_v, r0, r1, r2, acc_sh,
            g0, g1, g2, s0, s1, s2):
    rows = (r0, r1, r2)
    gsem = (g0, g1, g2)
    ssem = (s0, s1, s2)
    cid = lax.axis_index("c")
    sid = lax.axis_index("s")
    wid = sid * NC + cid

    pltpu.sync_copy(row_hbm.at[wid], row_v)
    pltpu.sync_copy(col_hbm.at[wid], col_v)

    def ridx(k):
        return row_v.at[pl.ds(pl.multiple_of(k * C, 8), C)]

    def cidx(k):
        return col_v.at[pl.ds(pl.multiple_of(k * C, 8), C)]

    _zero_vmem(r0, C, D)

    def zpiece(t, _):
        off = pl.multiple_of(sid * SLAB + t * 80, 8)
        pltpu.sync_copy(r0, acc_sh.at[pl.ds(off, 80)])
        return 0

    lax.fori_loop(0, 7, zpiece, 0)
    tail = pl.multiple_of(sid * SLAB + 560, 8)

    @pl.when(sid == NS - 1)
    def _():
        pltpu.sync_copy(r0, acc_sh.at[pl.ds(tail, 80)])

    @pl.when(sid < NS - 1)
    def _():
        pltpu.sync_copy(r0.at[pl.ds(0, 64)], acc_sh.at[pl.ds(tail, 64)])

    plsc.subcore_barrier()

    def stage_a(j, slot):
        pltpu.make_async_copy(h_hbm.at[ridx(0)], rows[slot], gsem[slot]).wait()
        pltpu.async_copy(rows[slot], acc_sh.at[cidx(j)], ssem[slot], add=True)

    def stage_b(j, slot):
        pltpu.make_async_copy(h_hbm.at[ridx(0)], rows[slot], ssem[slot]).wait()
        pltpu.async_copy(h_hbm.at[ridx(j + 3)], rows[slot], gsem[slot])

    for i in range(3):
        pltpu.async_copy(h_hbm.at[ridx(i)], rows[i], gsem[i])
    stage_a(0, 0)
    stage_a(1, 1)
    stage_b(0, 0)
    stage_a(2, 2)
    stage_b(1, 1)

    def ring(m, _):
        k = 3 * m
        stage_a(k, 0)
        stage_b(k - 1, 2)
        stage_a(k + 1, 1)
        stage_b(k, 0)
        stage_a(k + 2, 2)
        stage_b(k + 1, 1)
        return 0

    lax.fori_loop(1, 41, ring, 0)

    def drain_s(slot):
        pltpu.make_async_copy(h_hbm.at[ridx(0)], rows[slot], ssem[slot]).wait()

    stage_a(NCH - 2, 0)
    drain_s(2)
    stage_a(NCH - 1, 1)
    drain_s(0)
    drain_s(1)
    plsc.subcore_barrier()

    def wpiece(t, _):
        off = pl.multiple_of(sid * SLAB + t * 80, 8)
        pltpu.sync_copy(acc_sh.at[pl.ds(off, 80)], r0)
        pltpu.sync_copy(r0, out_hbm.at[cid, pl.ds(off, 80)])
        return 0

    lax.fori_loop(0, 7, wpiece, 0)

    @pl.when(sid == NS - 1)
    def _():
        pltpu.sync_copy(acc_sh.at[pl.ds(tail, 80)], r1)
        pltpu.sync_copy(r1, out_hbm.at[cid, pl.ds(tail, 80)])

    @pl.when(sid < NS - 1)
    def _():
        pltpu.sync_copy(acc_sh.at[pl.ds(tail, 64)], r1.at[pl.ds(0, 64)])
        pltpu.sync_copy(r1.at[pl.ds(0, 64)], out_hbm.at[cid, pl.ds(tail, 64)])


def _dis_of(dp_ref):
    deg = dp_ref[0, :N, 0:1] + dp_ref[1, :N, 0:1]
    return lax.rsqrt(deg)


def _mm_body(x_ref, w_ref, b_ref, o_ref):
    h = lax.dot_general(x_ref[...], w_ref[...], (((1,), (1,)), ((), ())),
                        precision=lax.Precision.HIGHEST)
    o_ref[...] = h + b_ref[...]


def _scale_body(h_ref, dp_ref, o_ref):
    o_ref[...] = h_ref[...] * _dis_of(dp_ref)


def _l1_body(p_ref, w_ref, b_ref, dp_ref, o_ref):
    dis = _dis_of(dp_ref)
    u = (p_ref[0] + p_ref[1]) * dis
    h = lax.dot_general(u, w_ref[...], (((1,), (1,)), ((), ())),
                        precision=lax.Precision.HIGHEST)
    o_ref[...] = (h + b_ref[...]) * dis


def _fin_body(p_ref, dp_ref, o_ref):
    o_ref[...] = (p_ref[0] + p_ref[1]) * _dis_of(dp_ref)


_out_nd = jax.ShapeDtypeStruct((N, D), jnp.float32)
_tc_mm = pl.pallas_call(_mm_body, out_shape=_out_nd)
_tc_scale = pl.pallas_call(_scale_body, out_shape=_out_nd)
_tc_l1 = pl.pallas_call(_l1_body, out_shape=_out_nd)
_tc_fin = pl.pallas_call(_fin_body, out_shape=_out_nd)


def kernel(x, edge_index, W0, b0, W1, b1):
    row2 = edge_index[0].astype(jnp.int32).reshape(NW, EW)
    col2 = edge_index[1].astype(jnp.int32).reshape(NW, EW)
    row3 = edge_index[0].astype(jnp.int32).reshape(NW, NCHD, CD)
    b0r = b0.reshape(1, D)
    b1r = b1.reshape(1, D)

    degp = _sc_deg(row3)
    mm0 = _tc_mm(x, W0, b0r)
    h0 = _tc_scale(mm0, degp)
    p0 = _sc_msg(h0, row2, col2)
    h1 = _tc_l1(p0, W1, b1r, degp)
    p1 = _sc_msg(h1, row2, col2)
    return _tc_fin(p1, degp)
